# initial kernel scaffold (unmeasured)
import jax
import jax.numpy as jnp
from jax import lax
from jax.experimental import pallas as pl
from jax.experimental.pallas import tpu as pltpu

N_DEV = 4
M_CHUNK = 1024
SLAB_N = 1024
N_SLABS = 8


def kernel(x, w_mat):
    m, k_per = x.shape
    _, n = w_mat.shape

    def body(x_ref, w_ref, out_ref, send_buf, recv_buf, send_sems, recv_sems):
        s = pl.program_id(0)
        my = lax.axis_index("i")
        left = (my - 1) % N_DEV
        right = (my + 1) % N_DEV

        @pl.when(s == 0)
        def _():
            barrier_sem = pltpu.get_barrier_semaphore()
            for nbr in [left, right]:
                pl.semaphore_signal(
                    barrier_sem, inc=1,
                    device_id=(nbr,), device_id_type=pl.DeviceIdType.MESH,
                )
            pl.semaphore_wait(barrier_sem, 2)

        def partial(c):
            xc = x_ref[pl.ds(c * M_CHUNK, M_CHUNK), :]
            return jnp.dot(xc, w_ref[:, :], preferred_element_type=jnp.float32)

        send_buf[:, :] = partial((my - 1) % N_DEV)

        for h in range(N_DEV - 1):
            rdma = pltpu.make_async_remote_copy(
                src_ref=send_buf,
                dst_ref=recv_buf.at[h],
                send_sem=send_sems.at[h],
                recv_sem=recv_sems.at[h],
                device_id=(right,),
                device_id_type=pl.DeviceIdType.MESH,
            )
            rdma.start()
            c_in = (my - 2 - h) % N_DEV
            p = partial(c_in)
            rdma.wait()
            if h < N_DEV - 2:
                send_buf[:, :] = recv_buf[h] + p
            else:
                out_ref[:, :] = jnp.maximum(recv_buf[h] + p, 0.0)

    grid = (N_SLABS,)
    return pl.pallas_call(
        body,
        grid=grid,
        out_shape=jax.ShapeDtypeStruct((M_CHUNK, n), jnp.float32),
        in_specs=[
            pl.BlockSpec((m, k_per), lambda s: (0, 0)),
            pl.BlockSpec((k_per, SLAB_N), lambda s: (0, s)),
        ],
        out_specs=pl.BlockSpec((M_CHUNK, SLAB_N), lambda s: (0, s)),
        scratch_shapes=[
            pltpu.VMEM((M_CHUNK, SLAB_N), jnp.float32),
            pltpu.VMEM((N_DEV - 1, M_CHUNK, SLAB_N), jnp.float32),
            pltpu.SemaphoreType.DMA((N_DEV - 1,)),
            pltpu.SemaphoreType.DMA((N_DEV - 1,)),
        ],
        compiler_params=pltpu.CompilerParams(
            collective_id=0,
            dimension_semantics=("arbitrary",),
        ),
    )(x, w_mat)


# baseline (device time: 1186185 ns/iter reference)
import jax
import jax.numpy as jnp
from jax import lax
from jax.experimental import pallas as pl
from jax.experimental.pallas import tpu as pltpu

N_DEV = 4
M_CHUNK = 1024
SLAB_N = 1024
N_SLABS = 8


def kernel(x, w_mat):
    m, k_per = x.shape
    _, n = w_mat.shape

    def body(x_ref, w_ref, out_ref, send_buf, recv_buf, send_sems, recv_sems):
        s = pl.program_id(0)
        my = lax.axis_index("i")
        left = (my - 1) % N_DEV
        right = (my + 1) % N_DEV

        @pl.when(s == 0)
        def _():
            barrier_sem = pltpu.get_barrier_semaphore()
            for nbr in [left, right]:
                pl.semaphore_signal(
                    barrier_sem, inc=1,
                    device_id=(nbr,), device_id_type=pl.DeviceIdType.MESH,
                )
            pl.semaphore_wait(barrier_sem, 2)

        def partial(c):
            xc = x_ref[pl.ds(c * M_CHUNK, M_CHUNK), :]
            return jnp.dot(xc, w_ref[:, :], preferred_element_type=jnp.float32)

        send_buf[:, :] = partial((my - 1) % N_DEV)

        for h in range(N_DEV - 1):
            rdma = pltpu.make_async_remote_copy(
                src_ref=send_buf,
                dst_ref=recv_buf.at[h],
                send_sem=send_sems.at[h],
                recv_sem=recv_sems.at[h],
                device_id=(right,),
                device_id_type=pl.DeviceIdType.MESH,
            )
            rdma.start()
            c_in = (my - 2 - h) % N_DEV
            p = partial(c_in)
            rdma.wait()
            if h < N_DEV - 2:
                send_buf[:, :] = recv_buf[h] + p
            else:
                out_ref[:, :] = jnp.maximum(recv_buf[h] + p, 0.0)

    grid = (N_SLABS,)
    return pl.pallas_call(
        body,
        grid=grid,
        out_shape=jax.ShapeDtypeStruct((M_CHUNK, n), jnp.float32),
        in_specs=[
            pl.BlockSpec((m, k_per), lambda s: (0, 0)),
            pl.BlockSpec((k_per, SLAB_N), lambda s: (0, s)),
        ],
        out_specs=pl.BlockSpec((M_CHUNK, SLAB_N), lambda s: (0, s)),
        scratch_shapes=[
            pltpu.VMEM((M_CHUNK, SLAB_N), jnp.float32),
            pltpu.VMEM((N_DEV - 1, M_CHUNK, SLAB_N), jnp.float32),
            pltpu.SemaphoreType.DMA((N_DEV - 1,)),
            pltpu.SemaphoreType.DMA((N_DEV - 1,)),
        ],
        compiler_params=pltpu.CompilerParams(
            collective_id=0,
            dimension_semantics=("arbitrary",),
            vmem_limit_bytes=60 * 1024 * 1024,
        ),
    )(x, w_mat)


# device time: 647021 ns/iter; 1.8333x vs baseline; 1.8333x over previous
import jax
import jax.numpy as jnp
from jax import lax
from jax.experimental import pallas as pl
from jax.experimental.pallas import tpu as pltpu

N_DEV = 4
M_CHUNK = 1024
SLAB_N = 1024
HALF_N = SLAB_N // 2
N_SLABS = 8


def kernel(x, w_mat):
    m, k_per = x.shape
    _, n = w_mat.shape

    def body(x_ref, w_ref, out_ref,
             send_r, send_l, recv_r, recv_l, sems_r, sems_l,
             recv_sems_r, recv_sems_l):
        s = pl.program_id(0)
        my = lax.axis_index("i")
        left = (my - 1) % N_DEV
        right = (my + 1) % N_DEV

        @pl.when(s == 0)
        def _():
            barrier_sem = pltpu.get_barrier_semaphore()
            for nbr in [left, right]:
                pl.semaphore_signal(
                    barrier_sem, inc=1,
                    device_id=(nbr,), device_id_type=pl.DeviceIdType.MESH,
                )
            pl.semaphore_wait(barrier_sem, 2)

        def partial_r(c):
            xc = x_ref[pl.ds(c * M_CHUNK, M_CHUNK), :]
            return jnp.dot(xc, w_ref[:, :HALF_N],
                           preferred_element_type=jnp.float32)

        def partial_l(c):
            xc = x_ref[pl.ds(c * M_CHUNK, M_CHUNK), :]
            return jnp.dot(xc, w_ref[:, HALF_N:],
                           preferred_element_type=jnp.float32)

        send_r[:, :] = partial_r((my - 1) % N_DEV)
        send_l[:, :] = partial_l((my + 1) % N_DEV)

        for h in range(N_DEV - 1):
            rdma_r = pltpu.make_async_remote_copy(
                src_ref=send_r,
                dst_ref=recv_r.at[h],
                send_sem=sems_r.at[h],
                recv_sem=recv_sems_r.at[h],
                device_id=(right,),
                device_id_type=pl.DeviceIdType.MESH,
            )
            rdma_l = pltpu.make_async_remote_copy(
                src_ref=send_l,
                dst_ref=recv_l.at[h],
                send_sem=sems_l.at[h],
                recv_sem=recv_sems_l.at[h],
                device_id=(left,),
                device_id_type=pl.DeviceIdType.MESH,
            )
            rdma_r.start()
            rdma_l.start()
            p_r = partial_r((my - 2 - h) % N_DEV)
            p_l = partial_l((my + 2 + h) % N_DEV)
            rdma_r.wait()
            rdma_l.wait()
            if h < N_DEV - 2:
                send_r[:, :] = recv_r[h] + p_r
                send_l[:, :] = recv_l[h] + p_l
            else:
                out_ref[:, :HALF_N] = jnp.maximum(recv_r[h] + p_r, 0.0)
                out_ref[:, HALF_N:] = jnp.maximum(recv_l[h] + p_l, 0.0)

    grid = (N_SLABS,)
    return pl.pallas_call(
        body,
        grid=grid,
        out_shape=jax.ShapeDtypeStruct((M_CHUNK, n), jnp.float32),
        in_specs=[
            pl.BlockSpec((m, k_per), lambda s: (0, 0)),
            pl.BlockSpec((k_per, SLAB_N), lambda s: (0, s)),
        ],
        out_specs=pl.BlockSpec((M_CHUNK, SLAB_N), lambda s: (0, s)),
        scratch_shapes=[
            pltpu.VMEM((M_CHUNK, HALF_N), jnp.float32),
            pltpu.VMEM((M_CHUNK, HALF_N), jnp.float32),
            pltpu.VMEM((N_DEV - 1, M_CHUNK, HALF_N), jnp.float32),
            pltpu.VMEM((N_DEV - 1, M_CHUNK, HALF_N), jnp.float32),
            pltpu.SemaphoreType.DMA((N_DEV - 1,)),
            pltpu.SemaphoreType.DMA((N_DEV - 1,)),
            pltpu.SemaphoreType.DMA((N_DEV - 1,)),
            pltpu.SemaphoreType.DMA((N_DEV - 1,)),
        ],
        compiler_params=pltpu.CompilerParams(
            collective_id=0,
            dimension_semantics=("arbitrary",),
            vmem_limit_bytes=60 * 1024 * 1024,
        ),
    )(x, w_mat)


# device time: 382342 ns/iter; 3.1024x vs baseline; 1.6923x over previous
import jax
import jax.numpy as jnp
from jax import lax
from jax.experimental import pallas as pl
from jax.experimental.pallas import tpu as pltpu

N_DEV = 4
M_CHUNK = 1024
SLAB_N = 1024
HALF_N = SLAB_N // 2
N_SLABS = 8192 // SLAB_N


def kernel(x, w_mat):
    m, k_per = x.shape
    _, n = w_mat.shape

    def body(x_ref, w_ref, out_ref,
             send_r, send_l, recv_r, recv_l, sems_r, sems_l,
             recv_sems_r, recv_sems_l):
        s = pl.program_id(0)
        my = lax.axis_index("i")
        left = (my - 1) % N_DEV
        right = (my + 1) % N_DEV

        @pl.when(s == 0)
        def _():
            barrier_sem = pltpu.get_barrier_semaphore()
            for nbr in [left, right]:
                pl.semaphore_signal(
                    barrier_sem, inc=1,
                    device_id=(nbr,), device_id_type=pl.DeviceIdType.MESH,
                )
            pl.semaphore_wait(barrier_sem, 2)

        def partial_r(c):
            xc = x_ref[pl.ds(c * M_CHUNK, M_CHUNK), :]
            return jnp.dot(xc, w_ref[:, :HALF_N],
                           preferred_element_type=jnp.float32)

        def partial_l(c):
            xc = x_ref[pl.ds(c * M_CHUNK, M_CHUNK), :]
            return jnp.dot(xc, w_ref[:, HALF_N:],
                           preferred_element_type=jnp.float32)

        send_r[:, :] = partial_r((my - 1) % N_DEV).astype(jnp.bfloat16)
        send_l[:, :] = partial_l((my + 1) % N_DEV).astype(jnp.bfloat16)

        for h in range(N_DEV - 1):
            rdma_r = pltpu.make_async_remote_copy(
                src_ref=send_r,
                dst_ref=recv_r.at[h],
                send_sem=sems_r.at[h],
                recv_sem=recv_sems_r.at[h],
                device_id=(right,),
                device_id_type=pl.DeviceIdType.MESH,
            )
            rdma_l = pltpu.make_async_remote_copy(
                src_ref=send_l,
                dst_ref=recv_l.at[h],
                send_sem=sems_l.at[h],
                recv_sem=recv_sems_l.at[h],
                device_id=(left,),
                device_id_type=pl.DeviceIdType.MESH,
            )
            rdma_r.start()
            rdma_l.start()
            p_r = partial_r((my - 2 - h) % N_DEV)
            p_l = partial_l((my + 2 + h) % N_DEV)
            rdma_r.wait()
            rdma_l.wait()
            if h < N_DEV - 2:
                send_r[:, :] = (recv_r[h] + p_r).astype(jnp.bfloat16)
                send_l[:, :] = (recv_l[h] + p_l).astype(jnp.bfloat16)
            else:
                out_ref[:, :HALF_N] = jnp.maximum(recv_r[h] + p_r, 0.0)
                out_ref[:, HALF_N:] = jnp.maximum(recv_l[h] + p_l, 0.0)

    grid = (N_SLABS,)
    return pl.pallas_call(
        body,
        grid=grid,
        out_shape=jax.ShapeDtypeStruct((M_CHUNK, n), jnp.float32),
        in_specs=[
            pl.BlockSpec((m, k_per), lambda s: (0, 0)),
            pl.BlockSpec((k_per, SLAB_N), lambda s: (0, s)),
        ],
        out_specs=pl.BlockSpec((M_CHUNK, SLAB_N), lambda s: (0, s)),
        scratch_shapes=[
            pltpu.VMEM((M_CHUNK, HALF_N), jnp.bfloat16),
            pltpu.VMEM((M_CHUNK, HALF_N), jnp.bfloat16),
            pltpu.VMEM((N_DEV - 1, M_CHUNK, HALF_N), jnp.bfloat16),
            pltpu.VMEM((N_DEV - 1, M_CHUNK, HALF_N), jnp.bfloat16),
            pltpu.SemaphoreType.DMA((N_DEV - 1,)),
            pltpu.SemaphoreType.DMA((N_DEV - 1,)),
            pltpu.SemaphoreType.DMA((N_DEV - 1,)),
            pltpu.SemaphoreType.DMA((N_DEV - 1,)),
        ],
        compiler_params=pltpu.CompilerParams(
            collective_id=0,
            dimension_semantics=("arbitrary",),
            vmem_limit_bytes=64 * 1024 * 1024,
        ),
    )(x, w_mat)
